# baseline (speedup 1.0000x reference)
import functools

import jax
import jax.numpy as jnp
from jax import lax
from jax.experimental import pallas as pl
from jax.experimental.pallas import tpu as pltpu
from jax.experimental.pallas import tpu_sc as plsc

_NT = 2048
_NB = 64
_NF = 5
_NS = 512
_BLK = 768
_GO = 128
_SSH = 640

_C = (0.9996203753455158, -0.4866430640453205, 0.254622206847047,
      -0.0747361476617855)


def _log1p_poly(x):
    r = jnp.float32(_C[3])
    for i in range(2, -1, -1):
        r = r * x + jnp.float32(_C[i])
    return r * x


def _logaddexp16(a, b):
    m = jnp.maximum(a, b)
    d = jnp.minimum(a, b) - m
    return m + _log1p_poly(jnp.exp(d))


def _make_sc_kernel():
    mesh = plsc.VectorSubcoreMesh(core_axis_name="c", subcore_axis_name="s")

    @functools.partial(
        pl.kernel,
        mesh=mesh,
        compiler_params=pltpu.CompilerParams(needs_layout_passes=False),
        out_type=jax.ShapeDtypeStruct((_NB, _BLK - _GO), jnp.float32),
        scratch_types=[
            pltpu.VMEM((2 * _NT * _NF,), jnp.float32),
            pltpu.VMEM((2 * _SSH,), jnp.int32),
            pltpu.VMEM((2 * _BLK,), jnp.float32),
            pltpu.VMEM((2 * _BLK,), jnp.float32),
            pltpu.VMEM((2 * _NB,), jnp.int32),
        ],
    )
    def sc_ctc(x_hbm, seq_hbm, info_hbm, out_hbm, xv, sv, fa, fb, iv):
        wid = lax.axis_index("c") * 16 + lax.axis_index("s")
        pltpu.sync_copy(info_hbm, iv)

        def scalar_at(i):
            return jnp.max(plsc.load_gather(
                iv, [jnp.full((16,), i, dtype=jnp.int32)]))

        ba = scalar_at(2 * wid)
        bb = scalar_at(2 * wid + 1)
        la = scalar_at(_NB + 2 * wid)
        lb = scalar_at(_NB + 2 * wid + 1)

        nx = _NT * _NF
        pltpu.sync_copy(x_hbm.at[ba], xv.at[pl.ds(0, nx)])
        pltpu.sync_copy(x_hbm.at[bb], xv.at[pl.ds(nx, nx)])
        pltpu.sync_copy(seq_hbm.at[ba], sv.at[pl.ds(0, _SSH)])
        pltpu.sync_copy(seq_hbm.at[bb], sv.at[pl.ds(_SSH, _SSH)])

        neg = jnp.full((16,), -1e30, dtype=jnp.float32)
        for i in range(2 * _BLK // 16):
            fa[pl.ds(i * 16, 16)] = neg
            fb[pl.ds(i * 16, 16)] = neg
        init0 = jnp.where(lax.iota(jnp.int32, 16) == 0,
                          jnp.float32(0.0), jnp.float32(-1e30))
        fa[pl.ds(_GO, 16)] = init0
        fa[pl.ds(_BLK + _GO, 16)] = init0

        def one_step(t, prev, new):
            rem = jnp.int32(_NT - 1) - t
            fcap = (t + 1) // 16
            xb0 = jnp.full((16,), t * _NF, dtype=jnp.int32)
            xb1 = xb0 + nx
            blank0 = plsc.load_gather(xv, [xb0 + 4])
            blank1 = plsc.load_gather(xv, [xb1 + 4])
            loa = (jnp.maximum(la - rem, 0) // 16) * 16
            ta = jnp.minimum(fcap, la // 16) * 16 + 16 - loa
            lob = (jnp.maximum(lb - rem, 0) // 16) * 16
            tb = jnp.minimum(fcap, lb // 16) * 16 + 16 - lob
            total = ((ta + tb + 127) // 128) * 128
            ba_ = _GO + loa
            bb_ = _BLK + _GO + lob - ta
            sa_ = loa
            sb_ = _SSH + lob - ta

            @plsc.parallel_loop(0, total, step=16, unroll=8)
            def chunk(w):
                isb = w >= ta
                base = jnp.where(isb, bb_, ba_) + w
                soff = jnp.where(isb, sb_, sa_) + w
                xb = jnp.where(isb, xb1, xb0)
                blank = jnp.where(isb, blank1, blank0)
                stay = prev[pl.ds(base, 16)]
                move = prev[pl.ds(base - 1, 16)]
                sq = sv[pl.ds(soff, 16)]
                emit = plsc.load_gather(xv, [xb + sq])
                r = _logaddexp16(stay + blank, move + emit)
                new[pl.ds(base, 16)] = r

        def two_steps(i, carry):
            t0 = 2 * i
            one_step(t0, fa, fb)
            one_step(t0 + 1, fb, fa)
            return carry

        lax.fori_loop(0, _NT // 2, two_steps, 0)

        pltpu.sync_copy(fa.at[pl.ds(_GO, _BLK - _GO)], out_hbm.at[ba])
        pltpu.sync_copy(fa.at[pl.ds(_BLK + _GO, _BLK - _GO)],
                        out_hbm.at[bb])

    return sc_ctc


_SC_CTC = _make_sc_kernel()


@jax.jit
def kernel(x, seqs, seqlens):
    nt, nb, _ = x.shape
    ns = seqs.shape[1]
    xT = jnp.transpose(x, (1, 0, 2)).astype(jnp.float32).reshape(nb, -1)
    s32 = seqs.astype(jnp.int32)
    seqshift = jnp.zeros((nb, _SSH), jnp.int32).at[:, 1:1 + ns].set(s32)
    l32 = seqlens.astype(jnp.int32)
    order = jnp.argsort(l32)
    perm = jnp.stack([order[:nb // 2], jnp.flip(order[nb // 2:])],
                     axis=1).reshape(-1)
    info = jnp.concatenate([perm, l32[perm]])
    fwd = _SC_CTC(xT, seqshift, info)
    vals = jnp.take_along_axis(fwd, l32[:, None], axis=1)
    return -vals / jnp.float32(nt)


# speedup vs baseline: 1.1568x; 1.1568x over previous
import functools

import jax
import jax.numpy as jnp
from jax import lax
from jax.experimental import pallas as pl
from jax.experimental.pallas import tpu as pltpu
from jax.experimental.pallas import tpu_sc as plsc

_NT = 2048
_NB = 64
_NF = 5
_NS = 512
_BLK = 768
_GO = 128
_SSH = 640

_C = (0.9996203753455158, -0.4866430640453205, 0.254622206847047,
      -0.0747361476617855)


def _log1p_poly(x):
    r = jnp.float32(_C[3])
    for i in range(2, -1, -1):
        r = r * x + jnp.float32(_C[i])
    return r * x


def _logaddexp16(a, b):
    m = jnp.maximum(a, b)
    d = jnp.minimum(a, b) - m
    return m + _log1p_poly(jnp.exp(d))


def _make_sc_kernel():
    mesh = plsc.VectorSubcoreMesh(core_axis_name="c", subcore_axis_name="s")

    @functools.partial(
        pl.kernel,
        mesh=mesh,
        compiler_params=pltpu.CompilerParams(needs_layout_passes=False),
        out_type=jax.ShapeDtypeStruct((_NB, _BLK - _GO), jnp.float32),
        scratch_types=[
            pltpu.VMEM((2 * _NT * _NF,), jnp.float32),
            pltpu.VMEM((2 * _SSH,), jnp.int32),
            pltpu.VMEM((2 * _BLK,), jnp.float32),
            pltpu.VMEM((2 * _BLK,), jnp.float32),
            pltpu.VMEM((2 * _NB,), jnp.int32),
        ],
    )
    def sc_ctc(x_hbm, seq_hbm, info_hbm, out_hbm, xv, sv, fa, fb, iv):
        wid = lax.axis_index("c") * 16 + lax.axis_index("s")
        pltpu.sync_copy(info_hbm, iv)

        def scalar_at(i):
            return jnp.max(plsc.load_gather(
                iv, [jnp.full((16,), i, dtype=jnp.int32)]))

        ba = scalar_at(2 * wid)
        bb = scalar_at(2 * wid + 1)
        la = scalar_at(_NB + 2 * wid)
        lb = scalar_at(_NB + 2 * wid + 1)

        nx = _NT * _NF
        pltpu.sync_copy(x_hbm.at[ba], xv.at[pl.ds(0, nx)])
        pltpu.sync_copy(x_hbm.at[bb], xv.at[pl.ds(nx, nx)])
        pltpu.sync_copy(seq_hbm.at[ba], sv.at[pl.ds(0, _SSH)])
        pltpu.sync_copy(seq_hbm.at[bb], sv.at[pl.ds(_SSH, _SSH)])

        iot = lax.iota(jnp.int32, 16)
        iot5 = 5 * iot + 4

        def blank_sum(bi):
            def bsum(i, acc):
                idx = jnp.full((16,), bi * nx + 80 * i, dtype=jnp.int32)
                return acc + plsc.load_gather(xv, [idx + iot5])
            acc = lax.fori_loop(0, _NT // 16, bsum,
                                jnp.zeros((16,), jnp.float32))
            return jnp.sum(acc)

        tot_a = blank_sum(0)
        tot_b = blank_sum(1)

        cidx = []
        for k in range(5):
            s = iot + k
            q = (3 * k + (s >= 5).astype(jnp.int32)
                 + (s >= 10).astype(jnp.int32) + (s >= 15).astype(jnp.int32))
            cidx.append(q * 5 + 4)

        @plsc.parallel_loop(0, 2 * nx, step=80, unroll=2)
        def xform(goff):
            gv = jnp.full((16,), goff, dtype=jnp.int32)
            vals = [xv[pl.ds(goff + 16 * k, 16)] for k in range(5)]
            bls = [plsc.load_gather(xv, [gv + cidx[k]]) for k in range(5)]
            for k in range(5):
                xv[pl.ds(goff + 16 * k, 16)] = vals[k] - bls[k]

        neg = jnp.full((16,), -1e30, dtype=jnp.float32)
        for i in range(2 * _BLK // 16):
            fa[pl.ds(i * 16, 16)] = neg
            fb[pl.ds(i * 16, 16)] = neg
        init0 = jnp.where(lax.iota(jnp.int32, 16) == 0,
                          jnp.float32(0.0), jnp.float32(-1e30))
        fa[pl.ds(_GO, 16)] = init0
        fa[pl.ds(_BLK + _GO, 16)] = init0

        def one_step(t, prev, new):
            rem = jnp.int32(_NT - 1) - t
            fcap = (t + 1) // 16
            xb0 = jnp.full((16,), t * _NF, dtype=jnp.int32)
            xb1 = xb0 + nx
            loa = (jnp.maximum(la - rem, 0) // 16) * 16
            ta = jnp.minimum(fcap, la // 16) * 16 + 16 - loa
            lob = (jnp.maximum(lb - rem, 0) // 16) * 16
            tb = jnp.minimum(fcap, lb // 16) * 16 + 16 - lob
            total = ((ta + tb + 63) // 64) * 64
            ba_ = _GO + loa
            bb_ = _BLK + _GO + lob - ta
            sa_ = loa
            sb_ = _SSH + lob - ta

            @plsc.parallel_loop(0, total, step=16, unroll=4)
            def chunk(w):
                isb = w >= ta
                base = jnp.where(isb, bb_, ba_) + w
                soff = jnp.where(isb, sb_, sa_) + w
                xb = jnp.where(isb, xb1, xb0)
                stay = prev[pl.ds(base, 16)]
                move = prev[pl.ds(base - 1, 16)]
                sq = sv[pl.ds(soff, 16)]
                emit = plsc.load_gather(xv, [xb + sq])
                r = _logaddexp16(stay, move + emit)
                new[pl.ds(base, 16)] = r

        def two_steps(i, carry):
            t0 = 2 * i
            one_step(t0, fa, fb)
            one_step(t0 + 1, fb, fa)
            return carry

        lax.fori_loop(0, _NT // 2, two_steps, 0)

        @plsc.parallel_loop(0, _BLK - _GO, step=16, unroll=4)
        def addtot(off):
            fa[pl.ds(_GO + off, 16)] = fa[pl.ds(_GO + off, 16)] + tot_a
            fa[pl.ds(_BLK + _GO + off, 16)] = (
                fa[pl.ds(_BLK + _GO + off, 16)] + tot_b)

        pltpu.sync_copy(fa.at[pl.ds(_GO, _BLK - _GO)], out_hbm.at[ba])
        pltpu.sync_copy(fa.at[pl.ds(_BLK + _GO, _BLK - _GO)],
                        out_hbm.at[bb])

    return sc_ctc


_SC_CTC = _make_sc_kernel()


@jax.jit
def kernel(x, seqs, seqlens):
    nt, nb, _ = x.shape
    ns = seqs.shape[1]
    xT = jnp.transpose(x, (1, 0, 2)).astype(jnp.float32).reshape(nb, -1)
    s32 = seqs.astype(jnp.int32)
    seqshift = jnp.zeros((nb, _SSH), jnp.int32).at[:, 1:1 + ns].set(s32)
    l32 = seqlens.astype(jnp.int32)
    order = jnp.argsort(l32)
    perm = jnp.stack([order[:nb // 2], jnp.flip(order[nb // 2:])],
                     axis=1).reshape(-1)
    info = jnp.concatenate([perm, l32[perm]])
    fwd = _SC_CTC(xT, seqshift, info)
    vals = jnp.take_along_axis(fwd, l32[:, None], axis=1)
    return -vals / jnp.float32(nt)


# speedup vs baseline: 1.1663x; 1.0082x over previous
import functools

import jax
import jax.numpy as jnp
from jax import lax
from jax.experimental import pallas as pl
from jax.experimental.pallas import tpu as pltpu
from jax.experimental.pallas import tpu_sc as plsc

_NT = 2048
_NB = 64
_NF = 5
_NS = 512
_BLK = 768
_GO = 128
_SSH = 640

_C = (0.9996203753455158, -0.4866430640453205, 0.254622206847047,
      -0.0747361476617855)


def _log1p_poly(x):
    r = jnp.float32(_C[3])
    for i in range(2, -1, -1):
        r = r * x + jnp.float32(_C[i])
    return r * x


def _logaddexp16(a, b):
    m = jnp.maximum(a, b)
    d = jnp.minimum(a, b) - m
    return m + _log1p_poly(jnp.exp(d))


def _make_sc_kernel():
    mesh = plsc.VectorSubcoreMesh(core_axis_name="c", subcore_axis_name="s")

    @functools.partial(
        pl.kernel,
        mesh=mesh,
        compiler_params=pltpu.CompilerParams(needs_layout_passes=False),
        out_type=jax.ShapeDtypeStruct((_NB, _BLK - _GO), jnp.float32),
        scratch_types=[
            pltpu.VMEM((2 * _NT * _NF,), jnp.float32),
            pltpu.VMEM((2 * _BLK,), jnp.int32),
            pltpu.VMEM((2 * _BLK,), jnp.float32),
            pltpu.VMEM((2 * _BLK,), jnp.float32),
            pltpu.VMEM((2 * _NB,), jnp.int32),
        ],
    )
    def sc_ctc(x_hbm, seq_hbm, info_hbm, out_hbm, xv, sv, fa, fb, iv):
        wid = lax.axis_index("c") * 16 + lax.axis_index("s")
        pltpu.sync_copy(info_hbm, iv)

        def scalar_at(i):
            return jnp.max(plsc.load_gather(
                iv, [jnp.full((16,), i, dtype=jnp.int32)]))

        ba = scalar_at(2 * wid)
        bb = scalar_at(2 * wid + 1)
        la = scalar_at(_NB + 2 * wid)
        lb = scalar_at(_NB + 2 * wid + 1)

        nx = _NT * _NF
        pltpu.sync_copy(x_hbm.at[ba], xv.at[pl.ds(0, nx)])
        pltpu.sync_copy(x_hbm.at[bb], xv.at[pl.ds(nx, nx)])
        pltpu.sync_copy(seq_hbm.at[ba], sv.at[pl.ds(0, _SSH)])
        pltpu.sync_copy(seq_hbm.at[bb], sv.at[pl.ds(_BLK, _SSH)])

        iot = lax.iota(jnp.int32, 16)
        iot5 = 5 * iot + 4

        def blank_sum(bi):
            def bsum(i, acc):
                idx = jnp.full((16,), bi * nx + 80 * i, dtype=jnp.int32)
                return acc + plsc.load_gather(xv, [idx + iot5])
            acc = lax.fori_loop(0, _NT // 16, bsum,
                                jnp.zeros((16,), jnp.float32))
            return jnp.sum(acc)

        tot_a = blank_sum(0)
        tot_b = blank_sum(1)

        cidx = []
        for k in range(5):
            s = iot + k
            q = (3 * k + (s >= 5).astype(jnp.int32)
                 + (s >= 10).astype(jnp.int32) + (s >= 15).astype(jnp.int32))
            cidx.append(q * 5 + 4)

        @plsc.parallel_loop(0, 2 * nx, step=80, unroll=2)
        def xform(goff):
            gv = jnp.full((16,), goff, dtype=jnp.int32)
            vals = [xv[pl.ds(goff + 16 * k, 16)] for k in range(5)]
            bls = [plsc.load_gather(xv, [gv + cidx[k]]) for k in range(5)]
            for k in range(5):
                xv[pl.ds(goff + 16 * k, 16)] = vals[k] - bls[k]

        neg = jnp.full((16,), -1e30, dtype=jnp.float32)
        for i in range(2 * _BLK // 16):
            fa[pl.ds(i * 16, 16)] = neg
            fb[pl.ds(i * 16, 16)] = neg
        init0 = jnp.where(lax.iota(jnp.int32, 16) == 0,
                          jnp.float32(0.0), jnp.float32(-1e30))
        fa[pl.ds(_GO, 16)] = init0
        fa[pl.ds(_BLK + _GO, 16)] = init0

        def one_step(t, prev, new):
            rem = jnp.int32(_NT - 1) - t
            fcap = (t + 1) // 16
            xb0 = jnp.full((16,), t * _NF, dtype=jnp.int32)
            xb1 = xb0 + nx
            loa = (jnp.maximum(la - rem, 0) // 16) * 16
            ta = jnp.minimum(fcap, la // 16) * 16 + 16 - loa
            lob = (jnp.maximum(lb - rem, 0) // 16) * 16
            tb = jnp.minimum(fcap, lb // 16) * 16 + 16 - lob
            total = ((ta + tb + 63) // 64) * 64
            ba_ = _GO + loa
            bb_ = _BLK + _GO + lob - ta

            @plsc.parallel_loop(0, total, step=16, unroll=4)
            def chunk(w):
                isb = w >= ta
                base = jnp.where(isb, bb_, ba_) + w
                xb = jnp.where(isb, xb1, xb0)
                stay = prev[pl.ds(base, 16)]
                move = prev[pl.ds(base - 1, 16)]
                sq = sv[pl.ds(base - _GO, 16)]
                emit = plsc.load_gather(xv, [xb + sq])
                r = _logaddexp16(stay, move + emit)
                new[pl.ds(base, 16)] = r

        def two_steps(i, carry):
            t0 = 2 * i
            one_step(t0, fa, fb)
            one_step(t0 + 1, fb, fa)
            return carry

        lax.fori_loop(0, _NT // 2, two_steps, 0)

        @plsc.parallel_loop(0, _BLK - _GO, step=16, unroll=4)
        def addtot(off):
            fa[pl.ds(_GO + off, 16)] = fa[pl.ds(_GO + off, 16)] + tot_a
            fa[pl.ds(_BLK + _GO + off, 16)] = (
                fa[pl.ds(_BLK + _GO + off, 16)] + tot_b)

        pltpu.sync_copy(fa.at[pl.ds(_GO, _BLK - _GO)], out_hbm.at[ba])
        pltpu.sync_copy(fa.at[pl.ds(_BLK + _GO, _BLK - _GO)],
                        out_hbm.at[bb])

    return sc_ctc


_SC_CTC = _make_sc_kernel()


@jax.jit
def kernel(x, seqs, seqlens):
    nt, nb, _ = x.shape
    ns = seqs.shape[1]
    xT = jnp.transpose(x, (1, 0, 2)).astype(jnp.float32).reshape(nb, -1)
    s32 = seqs.astype(jnp.int32)
    seqshift = jnp.zeros((nb, _SSH), jnp.int32).at[:, 1:1 + ns].set(s32)
    l32 = seqlens.astype(jnp.int32)
    order = jnp.argsort(l32)
    perm = jnp.stack([order[:nb // 2], jnp.flip(order[nb // 2:])],
                     axis=1).reshape(-1)
    info = jnp.concatenate([perm, l32[perm]])
    fwd = _SC_CTC(xT, seqshift, info)
    vals = jnp.take_along_axis(fwd, l32[:, None], axis=1)
    return -vals / jnp.float32(nt)


# speedup vs baseline: 1.1968x; 1.0262x over previous
import functools

import jax
import jax.numpy as jnp
from jax import lax
from jax.experimental import pallas as pl
from jax.experimental.pallas import tpu as pltpu
from jax.experimental.pallas import tpu_sc as plsc

_NT = 2048
_NB = 64
_NF = 5
_NS = 512
_BLK = 768
_GO = 128
_SSH = 640

_C = (0.9996203753455158, -0.4866430640453205, 0.254622206847047,
      -0.0747361476617855)


def _log1p_poly(x):
    r = jnp.float32(_C[3])
    for i in range(2, -1, -1):
        r = r * x + jnp.float32(_C[i])
    return r * x


def _logaddexp16(a, b):
    m = jnp.maximum(a, b)
    d = jnp.minimum(a, b) - m
    return m + _log1p_poly(jnp.exp(d))


def _make_sc_kernel():
    mesh = plsc.VectorSubcoreMesh(core_axis_name="c", subcore_axis_name="s")

    @functools.partial(
        pl.kernel,
        mesh=mesh,
        compiler_params=pltpu.CompilerParams(needs_layout_passes=False),
        out_type=jax.ShapeDtypeStruct((_NB, _BLK - _GO), jnp.float32),
        scratch_types=[
            pltpu.VMEM((2 * _NT * _NF,), jnp.float32),
            pltpu.VMEM((2 * _BLK,), jnp.int32),
            pltpu.VMEM((2 * _BLK,), jnp.float32),
            pltpu.VMEM((2 * _BLK,), jnp.float32),
            pltpu.VMEM((2 * _NB,), jnp.int32),
        ],
    )
    def sc_ctc(x_hbm, seq_hbm, info_hbm, out_hbm, xv, sv, fa, fb, iv):
        wid = lax.axis_index("c") * 16 + lax.axis_index("s")
        pltpu.sync_copy(info_hbm, iv)

        def scalar_at(i):
            return jnp.max(plsc.load_gather(
                iv, [jnp.full((16,), i, dtype=jnp.int32)]))

        ba = scalar_at(2 * wid)
        bb = scalar_at(2 * wid + 1)
        la = scalar_at(_NB + 2 * wid)
        lb = scalar_at(_NB + 2 * wid + 1)

        nx = _NT * _NF
        pltpu.sync_copy(x_hbm.at[ba], xv.at[pl.ds(0, nx)])
        pltpu.sync_copy(x_hbm.at[bb], xv.at[pl.ds(nx, nx)])
        pltpu.sync_copy(seq_hbm.at[ba], sv.at[pl.ds(0, _SSH)])
        pltpu.sync_copy(seq_hbm.at[bb], sv.at[pl.ds(_BLK, _SSH)])

        iot = lax.iota(jnp.int32, 16)
        iot5 = 5 * iot + 4

        def blank_sum(bi):
            def bsum(i, acc):
                idx = jnp.full((16,), bi * nx + 80 * i, dtype=jnp.int32)
                return acc + plsc.load_gather(xv, [idx + iot5])
            acc = lax.fori_loop(0, _NT // 16, bsum,
                                jnp.zeros((16,), jnp.float32))
            return jnp.sum(acc)

        tot_a = blank_sum(0)
        tot_b = blank_sum(1)

        cidx = []
        for k in range(5):
            s = iot + k
            q = (3 * k + (s >= 5).astype(jnp.int32)
                 + (s >= 10).astype(jnp.int32) + (s >= 15).astype(jnp.int32))
            cidx.append(q * 5 + 4)

        @plsc.parallel_loop(0, 2 * nx, step=80, unroll=2)
        def xform(goff):
            gv = jnp.full((16,), goff, dtype=jnp.int32)
            vals = [xv[pl.ds(goff + 16 * k, 16)] for k in range(5)]
            bls = [plsc.load_gather(xv, [gv + cidx[k]]) for k in range(5)]
            for k in range(5):
                xv[pl.ds(goff + 16 * k, 16)] = vals[k] - bls[k]

        neg = jnp.full((16,), -1e30, dtype=jnp.float32)
        for i in range(2 * _BLK // 16):
            fa[pl.ds(i * 16, 16)] = neg
            fb[pl.ds(i * 16, 16)] = neg
        init0 = jnp.where(lax.iota(jnp.int32, 16) == 0,
                          jnp.float32(0.0), jnp.float32(-1e30))
        fa[pl.ds(_GO, 16)] = init0
        fa[pl.ds(_BLK + _GO, 16)] = init0

        def one_step(t, prev, new):
            rem = jnp.int32(_NT - 1) - t
            fcap = (t + 1) // 16
            xb0 = jnp.full((16,), t * _NF, dtype=jnp.int32)
            loa = (jnp.maximum(la - rem, 0) // 16) * 16
            ta = jnp.minimum(fcap, la // 16) * 16 + 16 - loa
            lob = (jnp.maximum(lb - rem, 0) // 16) * 16
            tb = jnp.minimum(fcap, lb // 16) * 16 + 16 - lob
            total = ((ta + tb + 63) // 64) * 64
            ba_ = _GO + loa
            bb_ = _BLK + _GO + lob - ta

            @plsc.parallel_loop(0, total, step=16, unroll=4)
            def chunk(w):
                isb = w >= ta
                base = jnp.where(isb, bb_, ba_) + w
                stay = prev[pl.ds(base, 16)]
                move = prev[pl.ds(base - 1, 16)]
                sq = sv[pl.ds(base - _GO, 16)]
                emit = plsc.load_gather(xv, [xb0 + sq])
                r = _logaddexp16(stay, move + emit)
                new[pl.ds(base, 16)] = r

        def two_steps(i, carry):
            t0 = 2 * i
            one_step(t0, fa, fb)
            one_step(t0 + 1, fb, fa)
            return carry

        lax.fori_loop(0, _NT // 2, two_steps, 0)

        @plsc.parallel_loop(0, _BLK - _GO, step=16, unroll=4)
        def addtot(off):
            fa[pl.ds(_GO + off, 16)] = fa[pl.ds(_GO + off, 16)] + tot_a
            fa[pl.ds(_BLK + _GO + off, 16)] = (
                fa[pl.ds(_BLK + _GO + off, 16)] + tot_b)

        pltpu.sync_copy(fa.at[pl.ds(_GO, _BLK - _GO)], out_hbm.at[ba])
        pltpu.sync_copy(fa.at[pl.ds(_BLK + _GO, _BLK - _GO)],
                        out_hbm.at[bb])

    return sc_ctc


_SC_CTC = _make_sc_kernel()


@jax.jit
def kernel(x, seqs, seqlens):
    nt, nb, _ = x.shape
    ns = seqs.shape[1]
    xT = jnp.transpose(x, (1, 0, 2)).astype(jnp.float32).reshape(nb, -1)
    s32 = seqs.astype(jnp.int32)
    l32 = seqlens.astype(jnp.int32)
    order = jnp.argsort(l32)
    perm = jnp.stack([order[:nb // 2], jnp.flip(order[nb // 2:])],
                     axis=1).reshape(-1)
    info = jnp.concatenate([perm, l32[perm]])
    role = jnp.zeros((nb,), jnp.int32).at[perm[1::2]].set(1)
    seqshift = jnp.zeros((nb, _SSH), jnp.int32).at[:, 1:1 + ns].set(s32)
    seqshift = seqshift + role[:, None] * jnp.int32(_NT * _NF)
    fwd = _SC_CTC(xT, seqshift, info)
    vals = jnp.take_along_axis(fwd, l32[:, None], axis=1)
    return -vals / jnp.float32(nt)
